# baseline (speedup 1.0000x reference)
import jax
import jax.numpy as jnp
from jax import lax
from jax.experimental import pallas as pl
from jax.experimental.pallas import tpu as pltpu
from jax.experimental.pallas import tpu_sc as plsc

N = 1048576
K = 32
L = 65536

NC = 2
NS = 16
NW = NC * NS

CL = 512
CLK = CL * K
L_PER_W = L // NW
CHUNKS = L_PER_W // CL
RW = 16384
NDESC = CLK // RW


def _body(xf, idxf, w, out_hbm, dummy_hbm,
          tbuf0, tbuf1, gidx0, gidx1, data0, data1, wv,
          sem_i, sem_x, sem_o0, sem_o1):
    c = lax.axis_index("c")
    s = lax.axis_index("s")
    wid = s * NC + c

    tbufs = (tbuf0, tbuf1)
    gidxs = (gidx0, gidx1)
    datas = (data0, data1)

    pltpu.sync_copy(w, wv)
    w_lo = wv[pl.ds(0, 16)]
    w_hi = wv[pl.ds(16, 16)]
    iota = lax.iota(jnp.int32, 16)
    iota_l = iota * L

    def write_pattern(chunk, tbuf):
        l0 = wid * L_PER_W + chunk * CL

        def body(r, carry):
            base = l0 + r
            tbuf[pl.ds(2 * r * 16, 16)] = iota_l + base
            tbuf[pl.ds((2 * r + 1) * 16, 16)] = iota_l + (16 * L + base)
            return carry

        lax.fori_loop(0, CL, body, 0)

    def fire(src, idxbuf, dst, sem):
        def body(r, carry):
            pltpu.async_copy(src.at[idxbuf.at[pl.ds(r * RW, RW)]],
                             dst.at[pl.ds(r * RW, RW)], sem)
            return carry

        if NDESC == 1:
            pltpu.async_copy(src.at[idxbuf], dst, sem)
        else:
            lax.fori_loop(0, NDESC, body, 0)

    def scale(data):
        def body(g, carry):
            data[pl.ds(2 * g * 16, 16)] = data[pl.ds(2 * g * 16, 16)] * w_lo
            data[pl.ds((2 * g + 1) * 16, 16)] = (
                data[pl.ds((2 * g + 1) * 16, 16)] * w_hi)
            return carry

        lax.fori_loop(0, CL, body, 0)

    write_pattern(0, tbufs[0])
    fire(idxf, tbufs[0], gidxs[0], sem_i)
    for chunk in range(CHUNKS):
        p = chunk % 2
        if chunk + 1 < CHUNKS:
            write_pattern(chunk + 1, tbufs[1 - p])
        pltpu.make_async_copy(dummy_hbm, gidxs[p], sem_i).wait()
        fire(xf, gidxs[p], datas[p], sem_x)
        if chunk + 1 < CHUNKS:
            fire(idxf, tbufs[1 - p], gidxs[1 - p], sem_i)
        pltpu.make_async_copy(xf.at[pl.ds(0, CLK)], datas[p], sem_x).wait()
        sem_o = sem_o0 if p == 0 else sem_o1
        if chunk >= 2:
            pltpu.make_async_copy(xf.at[pl.ds(0, CLK)], datas[p],
                                  sem_o).wait()
        scale(datas[p])
        pltpu.async_copy(
            datas[p],
            out_hbm.at[pl.ds((wid * L_PER_W + chunk * CL) * K, CLK)],
            sem_o)
    pltpu.make_async_copy(xf.at[pl.ds(0, CLK)], datas[0], sem_o0).wait()
    pltpu.make_async_copy(xf.at[pl.ds(0, CLK)], datas[1], sem_o1).wait()


def kernel(x, idx, weight):
    mesh = plsc.VectorSubcoreMesh(core_axis_name="c", subcore_axis_name="s")
    out, _ = pl.kernel(
        _body,
        out_type=(
            jax.ShapeDtypeStruct((L * K,), jnp.float32),
            jax.ShapeDtypeStruct((CLK,), jnp.int32),
        ),
        mesh=mesh,
        scratch_types=[
            pltpu.VMEM((CLK,), jnp.int32),
            pltpu.VMEM((CLK,), jnp.int32),
            pltpu.VMEM((CLK,), jnp.int32),
            pltpu.VMEM((CLK,), jnp.int32),
            pltpu.VMEM((CLK,), jnp.float32),
            pltpu.VMEM((CLK,), jnp.float32),
            pltpu.VMEM((K,), jnp.float32),
            pltpu.SemaphoreType.DMA,
            pltpu.SemaphoreType.DMA,
            pltpu.SemaphoreType.DMA,
            pltpu.SemaphoreType.DMA,
        ],
    )(x, idx.reshape(-1), weight)
    return out


# speedup vs baseline: 1.0442x; 1.0442x over previous
import jax
import jax.numpy as jnp
from jax import lax
from jax.experimental import pallas as pl
from jax.experimental.pallas import tpu as pltpu
from jax.experimental.pallas import tpu_sc as plsc

N = 1048576
K = 32
L = 65536

NC = 2
NS = 16
NW = NC * NS

CL = 512
L_PER_W = L // NW
CHUNKS = L_PER_W // CL
TROWS = CL * K // 128
OUT_ROWS = L * K // 128
W_OUT_ROWS = L_PER_W * K // 128


def _body(xf, idxf, w, out_hbm, dummy_hbm,
          tbuf0, tbuf1, gidx0, gidx1, data0, data1, wv,
          sem_i, sem_x, sem_o0, sem_o1):
    c = lax.axis_index("c")
    s = lax.axis_index("s")
    wid = s * NC + c

    tbufs = (tbuf0, tbuf1)
    gidxs = (gidx0, gidx1)
    datas = (data0, data1)

    pltpu.sync_copy(w, wv)
    w_lo = wv[pl.ds(0, 16)]
    w_hi = wv[pl.ds(16, 16)]
    iota = lax.iota(jnp.int32, 16)
    iota_l = iota * L

    def write_pattern(chunk, tbuf):
        l0 = wid * L_PER_W + chunk * CL

        def body(r, carry):
            base = l0 + r * 4
            for v in range(8):
                tbuf[r, pl.ds(v * 16, 16)] = iota_l + (
                    (v % 2) * 16 * L + v // 2 + base)
            return carry

        lax.fori_loop(0, TROWS, body, 0)

    def fire_rows(src, idxbuf, dst, sem):
        def body(r, carry):
            pltpu.async_copy(src.at[idxbuf.at[r]], dst.at[r], sem)
            return carry

        lax.fori_loop(0, TROWS, body, 0)

    def scale(data):
        def body(r, carry):
            for v in range(8):
                wvec = w_lo if v % 2 == 0 else w_hi
                data[r, pl.ds(v * 16, 16)] = data[r, pl.ds(v * 16, 16)] * wvec
            return carry

        lax.fori_loop(0, TROWS, body, 0)

    write_pattern(0, tbufs[0])
    fire_rows(idxf, tbufs[0], gidxs[0], sem_i)
    for chunk in range(CHUNKS):
        p = chunk % 2
        if chunk + 1 < CHUNKS:
            write_pattern(chunk + 1, tbufs[1 - p])
        pltpu.make_async_copy(dummy_hbm, gidxs[p], sem_i).wait()
        fire_rows(xf, gidxs[p], datas[p], sem_x)
        if chunk + 1 < CHUNKS:
            fire_rows(idxf, tbufs[1 - p], gidxs[1 - p], sem_i)
        pltpu.make_async_copy(out_hbm.at[pl.ds(0, TROWS)], datas[p],
                              sem_x).wait()
        sem_o = sem_o0 if p == 0 else sem_o1
        if chunk >= 2:
            pltpu.make_async_copy(out_hbm.at[pl.ds(0, TROWS)], datas[p],
                                  sem_o).wait()
        scale(datas[p])
        pltpu.async_copy(
            datas[p],
            out_hbm.at[pl.ds(wid * W_OUT_ROWS + chunk * TROWS, TROWS)],
            sem_o)
    pltpu.make_async_copy(out_hbm.at[pl.ds(0, TROWS)], datas[0], sem_o0).wait()
    pltpu.make_async_copy(out_hbm.at[pl.ds(0, TROWS)], datas[1], sem_o1).wait()


def kernel(x, idx, weight):
    mesh = plsc.VectorSubcoreMesh(core_axis_name="c", subcore_axis_name="s")
    out2d, _ = pl.kernel(
        _body,
        out_type=(
            jax.ShapeDtypeStruct((OUT_ROWS, 128), jnp.float32),
            jax.ShapeDtypeStruct((TROWS, 128), jnp.int32),
        ),
        mesh=mesh,
        scratch_types=[
            pltpu.VMEM((TROWS, 128), jnp.int32),
            pltpu.VMEM((TROWS, 128), jnp.int32),
            pltpu.VMEM((TROWS, 128), jnp.int32),
            pltpu.VMEM((TROWS, 128), jnp.int32),
            pltpu.VMEM((TROWS, 128), jnp.float32),
            pltpu.VMEM((TROWS, 128), jnp.float32),
            pltpu.VMEM((K,), jnp.float32),
            pltpu.SemaphoreType.DMA,
            pltpu.SemaphoreType.DMA,
            pltpu.SemaphoreType.DMA,
            pltpu.SemaphoreType.DMA,
        ],
    )(x, idx.reshape(-1), weight)
    return out2d.reshape(-1)
